# baseline (speedup 1.0000x reference)
import jax
import jax.numpy as jnp
from jax import lax
from jax.experimental import pallas as pl
from jax.experimental.pallas import tpu as pltpu
from jax.experimental.pallas import tpu_sc as plsc

N = 10000
NPAD = 10240
D = 128
C = 16
E = 320000
NW = 32
CHUNK = 128
NCHUNK = 80
EPW = NCHUNK * CHUNK
EPAD = NW * EPW
RPT = NPAD // 16
NB = 4

_SC_MESH = plsc.VectorSubcoreMesh(core_axis_name="c", subcore_axis_name="s")
_SC_PARAMS = pltpu.CompilerParams(use_tc_tiling_on_sc=False,
                                  needs_layout_passes=False)


def _rsqrt16(d):
  i = plsc.bitcast(d, jnp.int32)
  y = plsc.bitcast(jnp.int32(0x5F3759DF) - (i >> 1), jnp.float32)
  for _ in range(3):
    y = y * (1.5 - 0.5 * d * y * y)
  return y


def _zero_acc(buf, acc_sh, r0):
  def zrow(r, _):
    buf[r, :] = jnp.zeros((C,), jnp.float32)
    return 0
  lax.fori_loop(0, CHUNK, zrow, 0)
  for k in range(RPT // CHUNK):
    pltpu.sync_copy(buf, acc_sh.at[pl.ds(r0 + k * CHUNK, CHUNK)])


def _edge_ring(g_sh, acc_sh, src_v, dst_v, w_v, gbuf, sbuf, gsem, ssem):
  for b in range(NB):
    pltpu.async_copy(g_sh.at[src_v.at[b]], gbuf[b], gsem[b])

  def group(gidx, _):
    for b in range(NB):
      c = gidx * NB + b
      pltpu.make_async_copy(g_sh.at[src_v.at[c]], gbuf[b], gsem[b]).wait()

      @pl.when(c >= NB)
      def _():
        pltpu.make_async_copy(sbuf[b], acc_sh.at[dst_v.at[c - NB]],
                              ssem[b]).wait()

      def mrow(j, _):
        wvec = w_v[c, pl.ds(j * 16, 16)]
        for r in range(16):
          rr = j * 16 + r
          sbuf[b][rr, :] = gbuf[b][rr, :] * wvec[r]
        return 0
      lax.fori_loop(0, CHUNK // 16, mrow, 0, unroll=True)
      pltpu.async_copy(sbuf[b], acc_sh.at[dst_v.at[c]], ssem[b], add=True)
      nxt = c + NB

      @pl.when(nxt < NCHUNK)
      def _():
        pltpu.async_copy(g_sh.at[src_v.at[nxt]], gbuf[b], gsem[b])
    return 0
  lax.fori_loop(0, NCHUNK // NB, group, 0)
  for b in range(NB):
    c = NCHUNK - NB + b
    pltpu.make_async_copy(sbuf[b], acc_sh.at[dst_v.at[c]], ssem[b]).wait()


_EDGE_SCRATCH = [
    pltpu.VMEM((NCHUNK, CHUNK), jnp.int32),
    pltpu.VMEM((NCHUNK, CHUNK), jnp.int32),
    pltpu.VMEM((NCHUNK, CHUNK), jnp.float32),
] + [pltpu.VMEM((CHUNK, C), jnp.float32)] * (2 * NB) \
  + [pltpu.SemaphoreType.DMA] * (2 * NB)


def _split_rs(rs):
  src_v, dst_v, w_v = rs[0:3]
  gbuf = rs[3:3 + NB]
  sbuf = rs[3 + NB:3 + 2 * NB]
  gsem = rs[3 + 2 * NB:3 + 3 * NB]
  ssem = rs[3 + 3 * NB:3 + 4 * NB]
  return src_v, dst_v, w_v, gbuf, sbuf, gsem, ssem


def _make_deg():
  def body(dst_hbm, w_hbm, out_hbm, acc_sh, *rs):
    _, dst_v, w_v, gbuf, sbuf, _, ssem = _split_rs(rs)
    cid = lax.axis_index("c")
    sid = lax.axis_index("s")
    wid = cid * 16 + sid
    pltpu.sync_copy(dst_hbm.at[wid], dst_v)
    pltpu.sync_copy(w_hbm.at[wid], w_v)
    r0 = sid * RPT
    _zero_acc(sbuf[0], acc_sh, r0)
    plsc.subcore_barrier()

    def group(gidx, _):
      for b in range(NB):
        c = gidx * NB + b

        @pl.when(c >= NB)
        def _():
          pltpu.make_async_copy(gbuf[b], acc_sh.at[dst_v.at[c - NB]],
                                ssem[b]).wait()

        def mrow(j, _):
          wvec = w_v[c, pl.ds(j * 16, 16)]
          for r in range(16):
            gbuf[b][j * 16 + r, :] = jnp.full((C,), 1.0, jnp.float32) * wvec[r]
          return 0
        lax.fori_loop(0, CHUNK // 16, mrow, 0)
        pltpu.async_copy(gbuf[b], acc_sh.at[dst_v.at[c]], ssem[b], add=True)
      return 0
    lax.fori_loop(0, NCHUNK // NB, group, 0)
    for b in range(NB):
      c = NCHUNK - NB + b
      pltpu.make_async_copy(gbuf[b], acc_sh.at[dst_v.at[c]], ssem[b]).wait()

    plsc.subcore_barrier()
    pltpu.sync_copy(acc_sh.at[pl.ds(r0, RPT)],
                    out_hbm.at[cid, pl.ds(r0, RPT)])

  return pl.kernel(
      body,
      out_type=jax.ShapeDtypeStruct((2, NPAD, C), jnp.float32),
      mesh=_SC_MESH,
      scratch_types=[pltpu.VMEM_SHARED((NPAD, C), jnp.float32)] + _EDGE_SCRATCH,
      compiler_params=_SC_PARAMS,
  )


def _make_hop1():
  def body(z_hbm, degp_hbm, src_hbm, dst_hbm, w_hbm,
           acc_out, g_out, dinv_out,
           g_sh, acc_sh, z_v, d0_v, d1_v, gst, dst_buf2, *rs):
    src_v, dst_v, w_v, gbuf, sbuf, gsem, ssem = _split_rs(rs)
    cid = lax.axis_index("c")
    sid = lax.axis_index("s")
    wid = cid * 16 + sid
    r0 = sid * RPT
    pltpu.sync_copy(src_hbm.at[wid], src_v)
    pltpu.sync_copy(dst_hbm.at[wid], dst_v)
    pltpu.sync_copy(w_hbm.at[wid], w_v)
    pltpu.sync_copy(z_hbm.at[pl.ds(sid * (RPT // 8), RPT // 8)], z_v)
    pltpu.sync_copy(degp_hbm.at[0, pl.ds(r0, RPT)], d0_v)
    pltpu.sync_copy(degp_hbm.at[1, pl.ds(r0, RPT)], d1_v)

    def prow(n, _):
      d = d0_v[n, :] + d1_v[n, :] + 1.0
      y = _rsqrt16(d)
      rr = n // 8
      l16 = (n % 8) * 16
      zv = z_v[rr, pl.ds(l16, 16)]
      dst_buf2[n, :] = y
      gst[n, :] = y * zv
      return 0
    lax.fori_loop(0, RPT, prow, 0)
    pltpu.sync_copy(gst, g_sh.at[pl.ds(r0, RPT)])

    @pl.when(cid == 0)
    def _():
      pltpu.sync_copy(gst, g_out.at[pl.ds(r0, RPT)])
      pltpu.sync_copy(dst_buf2, dinv_out.at[pl.ds(r0, RPT)])

    _zero_acc(sbuf[0], acc_sh, r0)
    plsc.subcore_barrier()
    _edge_ring(g_sh, acc_sh, src_v, dst_v, w_v, gbuf, sbuf, gsem, ssem)
    plsc.subcore_barrier()
    pltpu.sync_copy(acc_sh.at[pl.ds(r0, RPT)],
                    acc_out.at[cid, pl.ds(r0, RPT)])

  return pl.kernel(
      body,
      out_type=[jax.ShapeDtypeStruct((2, NPAD, C), jnp.float32),
                jax.ShapeDtypeStruct((NPAD, C), jnp.float32),
                jax.ShapeDtypeStruct((NPAD, C), jnp.float32)],
      mesh=_SC_MESH,
      scratch_types=[
          pltpu.VMEM_SHARED((NPAD, C), jnp.float32),
          pltpu.VMEM_SHARED((NPAD, C), jnp.float32),
          pltpu.VMEM((RPT // 8, 128), jnp.float32),
          pltpu.VMEM((RPT, C), jnp.float32),
          pltpu.VMEM((RPT, C), jnp.float32),
          pltpu.VMEM((RPT, C), jnp.float32),
          pltpu.VMEM((RPT, C), jnp.float32),
      ] + _EDGE_SCRATCH,
      compiler_params=_SC_PARAMS,
  )


def _make_hop2():
  def body(g1_hbm, dinv_hbm, acc1_hbm, src_hbm, dst_hbm, w_hbm,
           acc_out, g2_out,
           g_sh, acc_sh, g1_v, dv_v, a0_v, a1_v, gst, *rs):
    src_v, dst_v, w_v, gbuf, sbuf, gsem, ssem = _split_rs(rs)
    cid = lax.axis_index("c")
    sid = lax.axis_index("s")
    wid = cid * 16 + sid
    r0 = sid * RPT
    pltpu.sync_copy(src_hbm.at[wid], src_v)
    pltpu.sync_copy(dst_hbm.at[wid], dst_v)
    pltpu.sync_copy(w_hbm.at[wid], w_v)
    pltpu.sync_copy(g1_hbm.at[pl.ds(r0, RPT)], g1_v)
    pltpu.sync_copy(dinv_hbm.at[pl.ds(r0, RPT)], dv_v)
    pltpu.sync_copy(acc1_hbm.at[0, pl.ds(r0, RPT)], a0_v)
    pltpu.sync_copy(acc1_hbm.at[1, pl.ds(r0, RPT)], a1_v)

    def prow(n, _):
      y = dv_v[n, :]
      gst[n, :] = y * y * (a0_v[n, :] + a1_v[n, :] + g1_v[n, :])
      return 0
    lax.fori_loop(0, RPT, prow, 0)
    pltpu.sync_copy(gst, g_sh.at[pl.ds(r0, RPT)])

    @pl.when(cid == 0)
    def _():
      pltpu.sync_copy(gst, g2_out.at[pl.ds(r0, RPT)])

    _zero_acc(sbuf[0], acc_sh, r0)
    plsc.subcore_barrier()
    _edge_ring(g_sh, acc_sh, src_v, dst_v, w_v, gbuf, sbuf, gsem, ssem)
    plsc.subcore_barrier()
    pltpu.sync_copy(acc_sh.at[pl.ds(r0, RPT)],
                    acc_out.at[cid, pl.ds(r0, RPT)])

  return pl.kernel(
      body,
      out_type=[jax.ShapeDtypeStruct((2, NPAD, C), jnp.float32),
                jax.ShapeDtypeStruct((NPAD, C), jnp.float32)],
      mesh=_SC_MESH,
      scratch_types=[
          pltpu.VMEM_SHARED((NPAD, C), jnp.float32),
          pltpu.VMEM_SHARED((NPAD, C), jnp.float32),
          pltpu.VMEM((RPT, C), jnp.float32),
          pltpu.VMEM((RPT, C), jnp.float32),
          pltpu.VMEM((RPT, C), jnp.float32),
          pltpu.VMEM((RPT, C), jnp.float32),
          pltpu.VMEM((RPT, C), jnp.float32),
      ] + _EDGE_SCRATCH,
      compiler_params=_SC_PARAMS,
  )


_SC_DEG = _make_deg()
_SC_HOP1 = _make_hop1()
_SC_HOP2 = _make_hop2()


def _tc_z(x3, W1):
  def body(x_ref, w1_ref, z_ref):
    z_ref[pl.ds(N // 8, (NPAD - N) // 8), :] = jnp.zeros(
        ((NPAD - N) // 8, 128), jnp.float32)
    for j in range(8):
      xj = x_ref[:, j, :]
      zj = lax.dot_general(xj, w1_ref[:, :], (((1,), (1,)), ((), ())),
                           preferred_element_type=jnp.float32)
      z_ref[pl.ds(0, N // 8), pl.ds(j * 16, 16)] = zj

  return pl.pallas_call(
      body,
      out_shape=jax.ShapeDtypeStruct((NPAD // 8, 128), jnp.float32),
  )(x3, W1)


def _tc_final(acc2, g2, dinv, b1t, W2, b2t):
  def body(acc_ref, g_ref, dinv_ref, b1_ref, w2_ref, b2_ref, out_ref):
    h = dinv_ref[:, :] * (acc_ref[0] + acc_ref[1] + g_ref[:, :])
    a = jnp.maximum(h + b1_ref[:, :], 0.0)
    for j in range(8):
      aj = a[:, j * 16:(j + 1) * 16]
      yj = lax.dot_general(aj, w2_ref[:, :], (((1,), (1,)), ((), ())),
                           preferred_element_type=jnp.float32)
      out_ref[:, pl.ds(j * 16, 16)] = jax.nn.sigmoid(
          yj + b2_ref[:, pl.ds(j * 16, 16)])

  return pl.pallas_call(
      body,
      out_shape=jax.ShapeDtypeStruct((N // 8, 128), jnp.float32),
  )(acc2, g2, dinv, b1t, W2, b2t)


def kernel(x, edge_index, edge_weight, W1, b1, W2, b2):
  pad_e = EPAD - E
  ei_p = jnp.pad(edge_index.astype(jnp.int32), ((0, 0), (0, pad_e)),
                 constant_values=NPAD - 1)
  src_p = ei_p[0].reshape(NW, NCHUNK, CHUNK)
  dst_p = ei_p[1].reshape(NW, NCHUNK, CHUNK)
  w_p = jnp.pad(edge_weight.astype(jnp.float32),
                (0, pad_e)).reshape(NW, NCHUNK, CHUNK)
  x3 = x.reshape(N // 8, 8, D)
  b1t = jnp.tile(b1, 8).reshape(1, 128)
  b2t = jnp.tile(b2, 8).reshape(1, 128)

  degp = _SC_DEG(dst_p, w_p)
  z128 = _tc_z(x3, W1)
  acc1, g1, dinv = _SC_HOP1(z128, degp, src_p, dst_p, w_p)
  acc2, g2 = _SC_HOP2(g1, dinv, acc1, src_p, dst_p, w_p)
  out128 = _tc_final(acc2.reshape(2, NPAD // 8, 128)[:, :N // 8],
                     g2.reshape(NPAD // 8, 128)[:N // 8],
                     dinv.reshape(NPAD // 8, 128)[:N // 8],
                     b1t, W2, b2t)
  return out128.reshape(N, C)


# speedup vs baseline: 1.2309x; 1.2309x over previous
import jax
import jax.numpy as jnp
from jax import lax
from jax.experimental import pallas as pl
from jax.experimental.pallas import tpu as pltpu
from jax.experimental.pallas import tpu_sc as plsc

N = 10000
NPAD = 10240
D = 128
C = 16
E = 320000
NW = 32
CHUNK = 128
EROWS = E // CHUNK
NCK = 79
NCL = EROWS - 31 * NCK
NGROUP = 20
RPT = NPAD // 16
NB = 4

_SC_MESH = plsc.VectorSubcoreMesh(core_axis_name="c", subcore_axis_name="s")
_SC_PARAMS = pltpu.CompilerParams(use_tc_tiling_on_sc=False,
                                  needs_layout_passes=False)


def _rsqrt16(d):
  i = plsc.bitcast(d, jnp.int32)
  y = plsc.bitcast(jnp.int32(0x5F3759DF) - (i >> 1), jnp.float32)
  for _ in range(3):
    y = y * (1.5 - 0.5 * d * y * y)
  return y


def _zero_acc(buf, acc_sh, r0):
  def zrow(r, _):
    buf[r, :] = jnp.zeros((C,), jnp.float32)
    return 0
  lax.fori_loop(0, CHUNK, zrow, 0)
  for k in range(RPT // CHUNK):
    pltpu.sync_copy(buf, acc_sh.at[pl.ds(r0 + k * CHUNK, CHUNK)])


def _load_slabs(wid, refs_hbm, refs_v):
  @pl.when(wid < NW - 1)
  def _():
    for hbm, v in zip(refs_hbm, refs_v):
      pltpu.sync_copy(hbm.at[pl.ds(wid * NCK, NCK)], v)

  @pl.when(wid == NW - 1)
  def _():
    for hbm, v in zip(refs_hbm, refs_v):
      pltpu.sync_copy(hbm.at[pl.ds(wid * NCK, NCL)], v.at[pl.ds(0, NCL)])


def _edge_ring(nc, g_sh, acc_sh, src_v, dst_v, w_v, gbuf, sbuf, gsem, ssem):
  for b in range(NB):
    pltpu.async_copy(g_sh.at[src_v.at[b]], gbuf[b], gsem[b])

  def group(gidx, _):
    for b in range(NB):
      c = gidx * NB + b

      @pl.when(c < nc)
      def _():
        pltpu.make_async_copy(g_sh.at[src_v.at[c]], gbuf[b], gsem[b]).wait()

        @pl.when(c >= NB)
        def _():
          pltpu.make_async_copy(sbuf[b], acc_sh.at[dst_v.at[c - NB]],
                                ssem[b]).wait()

        def mrow(j, _):
          wvec = w_v[c, pl.ds(j * 16, 16)]
          for r in range(16):
            rr = j * 16 + r
            sbuf[b][rr, :] = gbuf[b][rr, :] * wvec[r]
          return 0
        lax.fori_loop(0, CHUNK // 16, mrow, 0)
        pltpu.async_copy(sbuf[b], acc_sh.at[dst_v.at[c]], ssem[b], add=True)
        nxt = c + NB

        @pl.when(nxt < nc)
        def _():
          pltpu.async_copy(g_sh.at[src_v.at[nxt]], gbuf[b], gsem[b])
    return 0
  lax.fori_loop(0, NGROUP, group, 0)
  for b, off in ((3, -4), (0, -3), (1, -2), (2, -1)):
    c = nc + off
    pltpu.make_async_copy(sbuf[b], acc_sh.at[dst_v.at[c]], ssem[b]).wait()


_EDGE_SCRATCH = [
    pltpu.VMEM((NCK, CHUNK), jnp.int32),
    pltpu.VMEM((NCK, CHUNK), jnp.int32),
    pltpu.VMEM((NCK, CHUNK), jnp.float32),
] + [pltpu.VMEM((CHUNK, C), jnp.float32)] * (2 * NB) \
  + [pltpu.SemaphoreType.DMA] * (2 * NB)


def _split_rs(rs):
  src_v, dst_v, w_v = rs[0:3]
  gbuf = rs[3:3 + NB]
  sbuf = rs[3 + NB:3 + 2 * NB]
  gsem = rs[3 + 2 * NB:3 + 3 * NB]
  ssem = rs[3 + 3 * NB:3 + 4 * NB]
  return src_v, dst_v, w_v, gbuf, sbuf, gsem, ssem


def _make_deg():
  def body(dst_hbm, w_hbm, out_hbm, acc_sh, *rs):
    _, dst_v, w_v, gbuf, sbuf, _, ssem = _split_rs(rs)
    cid = lax.axis_index("c")
    sid = lax.axis_index("s")
    wid = cid * 16 + sid
    nc = jnp.where(wid == NW - 1, NCL, NCK)
    _load_slabs(wid, (dst_hbm, w_hbm), (dst_v, w_v))
    r0 = sid * RPT
    _zero_acc(sbuf[0], acc_sh, r0)
    plsc.subcore_barrier()

    def group(gidx, _):
      for b in range(NB):
        c = gidx * NB + b

        @pl.when(c < nc)
        def _():
          @pl.when(c >= NB)
          def _():
            pltpu.make_async_copy(gbuf[b], acc_sh.at[dst_v.at[c - NB]],
                                  ssem[b]).wait()

          def mrow(j, _):
            wvec = w_v[c, pl.ds(j * 16, 16)]
            for r in range(16):
              gbuf[b][j * 16 + r, :] = jnp.full((C,), 1.0,
                                                jnp.float32) * wvec[r]
            return 0
          lax.fori_loop(0, CHUNK // 16, mrow, 0)
          pltpu.async_copy(gbuf[b], acc_sh.at[dst_v.at[c]], ssem[b], add=True)
      return 0
    lax.fori_loop(0, NGROUP, group, 0)
    for b, off in ((3, -4), (0, -3), (1, -2), (2, -1)):
      c = nc + off
      pltpu.make_async_copy(gbuf[b], acc_sh.at[dst_v.at[c]], ssem[b]).wait()

    plsc.subcore_barrier()
    pltpu.sync_copy(acc_sh.at[pl.ds(r0, RPT)],
                    out_hbm.at[cid, pl.ds(r0, RPT)])

  return pl.kernel(
      body,
      out_type=jax.ShapeDtypeStruct((2, NPAD, C), jnp.float32),
      mesh=_SC_MESH,
      scratch_types=[pltpu.VMEM_SHARED((NPAD, C), jnp.float32)] + _EDGE_SCRATCH,
      compiler_params=_SC_PARAMS,
  )


def _make_hop1():
  def body(z_hbm, degp_hbm, src_hbm, dst_hbm, w_hbm,
           acc_out, g_out, dinv_out,
           g_sh, acc_sh, z_v, d0_v, d1_v, gst, dvst, *rs):
    src_v, dst_v, w_v, gbuf, sbuf, gsem, ssem = _split_rs(rs)
    cid = lax.axis_index("c")
    sid = lax.axis_index("s")
    wid = cid * 16 + sid
    nc = jnp.where(wid == NW - 1, NCL, NCK)
    r0 = sid * RPT
    _load_slabs(wid, (src_hbm, dst_hbm, w_hbm), (src_v, dst_v, w_v))
    pltpu.sync_copy(z_hbm.at[pl.ds(sid * (RPT // 8), RPT // 8)], z_v)
    pltpu.sync_copy(degp_hbm.at[0, pl.ds(r0, RPT)], d0_v)
    pltpu.sync_copy(degp_hbm.at[1, pl.ds(r0, RPT)], d1_v)

    def prow(n, _):
      d = d0_v[n, :] + d1_v[n, :] + 1.0
      y = _rsqrt16(d)
      rr = n // 8
      l16 = (n % 8) * 16
      zv = z_v[rr, pl.ds(l16, 16)]
      dvst[n, :] = y
      gst[n, :] = y * zv
      return 0
    lax.fori_loop(0, RPT, prow, 0)
    pltpu.sync_copy(gst, g_sh.at[pl.ds(r0, RPT)])

    @pl.when(cid == 0)
    def _():
      pltpu.sync_copy(gst, g_out.at[pl.ds(r0, RPT)])
      pltpu.sync_copy(dvst, dinv_out.at[pl.ds(r0, RPT)])

    _zero_acc(sbuf[0], acc_sh, r0)
    plsc.subcore_barrier()
    _edge_ring(nc, g_sh, acc_sh, src_v, dst_v, w_v, gbuf, sbuf, gsem, ssem)
    plsc.subcore_barrier()
    pltpu.sync_copy(acc_sh.at[pl.ds(r0, RPT)],
                    acc_out.at[cid, pl.ds(r0, RPT)])

  return pl.kernel(
      body,
      out_type=[jax.ShapeDtypeStruct((2, NPAD, C), jnp.float32),
                jax.ShapeDtypeStruct((NPAD, C), jnp.float32),
                jax.ShapeDtypeStruct((NPAD, C), jnp.float32)],
      mesh=_SC_MESH,
      scratch_types=[
          pltpu.VMEM_SHARED((NPAD, C), jnp.float32),
          pltpu.VMEM_SHARED((NPAD, C), jnp.float32),
          pltpu.VMEM((RPT // 8, 128), jnp.float32),
          pltpu.VMEM((RPT, C), jnp.float32),
          pltpu.VMEM((RPT, C), jnp.float32),
          pltpu.VMEM((RPT, C), jnp.float32),
          pltpu.VMEM((RPT, C), jnp.float32),
      ] + _EDGE_SCRATCH,
      compiler_params=_SC_PARAMS,
  )


def _make_hop2():
  def body(g1_hbm, dinv_hbm, acc1_hbm, src_hbm, dst_hbm, w_hbm,
           acc_out, g2_out,
           g_sh, acc_sh, g1_v, dv_v, a0_v, a1_v, gst, *rs):
    src_v, dst_v, w_v, gbuf, sbuf, gsem, ssem = _split_rs(rs)
    cid = lax.axis_index("c")
    sid = lax.axis_index("s")
    wid = cid * 16 + sid
    nc = jnp.where(wid == NW - 1, NCL, NCK)
    r0 = sid * RPT
    _load_slabs(wid, (src_hbm, dst_hbm, w_hbm), (src_v, dst_v, w_v))
    pltpu.sync_copy(g1_hbm.at[pl.ds(r0, RPT)], g1_v)
    pltpu.sync_copy(dinv_hbm.at[pl.ds(r0, RPT)], dv_v)
    pltpu.sync_copy(acc1_hbm.at[0, pl.ds(r0, RPT)], a0_v)
    pltpu.sync_copy(acc1_hbm.at[1, pl.ds(r0, RPT)], a1_v)

    def prow(n, _):
      y = dv_v[n, :]
      gst[n, :] = y * y * (a0_v[n, :] + a1_v[n, :] + g1_v[n, :])
      return 0
    lax.fori_loop(0, RPT, prow, 0)
    pltpu.sync_copy(gst, g_sh.at[pl.ds(r0, RPT)])

    @pl.when(cid == 0)
    def _():
      pltpu.sync_copy(gst, g2_out.at[pl.ds(r0, RPT)])

    _zero_acc(sbuf[0], acc_sh, r0)
    plsc.subcore_barrier()
    _edge_ring(nc, g_sh, acc_sh, src_v, dst_v, w_v, gbuf, sbuf, gsem, ssem)
    plsc.subcore_barrier()
    pltpu.sync_copy(acc_sh.at[pl.ds(r0, RPT)],
                    acc_out.at[cid, pl.ds(r0, RPT)])

  return pl.kernel(
      body,
      out_type=[jax.ShapeDtypeStruct((2, NPAD, C), jnp.float32),
                jax.ShapeDtypeStruct((NPAD, C), jnp.float32)],
      mesh=_SC_MESH,
      scratch_types=[
          pltpu.VMEM_SHARED((NPAD, C), jnp.float32),
          pltpu.VMEM_SHARED((NPAD, C), jnp.float32),
          pltpu.VMEM((RPT, C), jnp.float32),
          pltpu.VMEM((RPT, C), jnp.float32),
          pltpu.VMEM((RPT, C), jnp.float32),
          pltpu.VMEM((RPT, C), jnp.float32),
          pltpu.VMEM((RPT, C), jnp.float32),
      ] + _EDGE_SCRATCH,
      compiler_params=_SC_PARAMS,
  )


_SC_DEG = _make_deg()
_SC_HOP1 = _make_hop1()
_SC_HOP2 = _make_hop2()


def _tc_z(x3, W1):
  def body(x_ref, w1_ref, z_ref):
    z_ref[pl.ds(N // 8, (NPAD - N) // 8), :] = jnp.zeros(
        ((NPAD - N) // 8, 128), jnp.float32)
    for j in range(8):
      xj = x_ref[:, j, :]
      zj = lax.dot_general(xj, w1_ref[:, :], (((1,), (1,)), ((), ())),
                           preferred_element_type=jnp.float32)
      z_ref[pl.ds(0, N // 8), pl.ds(j * 16, 16)] = zj

  return pl.pallas_call(
      body,
      out_shape=jax.ShapeDtypeStruct((NPAD // 8, 128), jnp.float32),
  )(x3, W1)


def _tc_final(acc2, g2, dinv, b1t, W2, b2t):
  def body(acc_ref, g_ref, dinv_ref, b1_ref, w2_ref, b2_ref, out_ref):
    h = dinv_ref[:, :] * (acc_ref[0] + acc_ref[1] + g_ref[:, :])
    a = jnp.maximum(h + b1_ref[:, :], 0.0)
    for j in range(8):
      aj = a[:, j * 16:(j + 1) * 16]
      yj = lax.dot_general(aj, w2_ref[:, :], (((1,), (1,)), ((), ())),
                           preferred_element_type=jnp.float32)
      out_ref[:, pl.ds(j * 16, 16)] = jax.nn.sigmoid(
          yj + b2_ref[:, pl.ds(j * 16, 16)])

  return pl.pallas_call(
      body,
      out_shape=jax.ShapeDtypeStruct((NPAD // 8, 128), jnp.float32),
  )(acc2, g2, dinv, b1t, W2, b2t)


def kernel(x, edge_index, edge_weight, W1, b1, W2, b2):
  src_p = edge_index[0].astype(jnp.int32).reshape(EROWS, CHUNK)
  dst_p = edge_index[1].astype(jnp.int32).reshape(EROWS, CHUNK)
  w_p = edge_weight.astype(jnp.float32).reshape(EROWS, CHUNK)
  x3 = x.reshape(N // 8, 8, D)
  b1t = jnp.tile(b1, 8).reshape(1, 128)
  b2t = jnp.tile(b2, 8).reshape(1, 128)

  degp = _SC_DEG(dst_p, w_p)
  z128 = _tc_z(x3, W1)
  acc1, g1, dinv = _SC_HOP1(z128, degp, src_p, dst_p, w_p)
  acc2, g2 = _SC_HOP2(g1, dinv, acc1, src_p, dst_p, w_p)
  out128 = _tc_final(acc2.reshape(2, NPAD // 8, 128),
                     g2.reshape(NPAD // 8, 128),
                     dinv.reshape(NPAD // 8, 128),
                     b1t, W2, b2t)
  return out128.reshape(NPAD, C)[:N]
